# baseline (speedup 1.0000x reference)
import functools

import jax
import jax.numpy as jnp
from jax import lax
from jax.experimental import pallas as pl
from jax.experimental.pallas import tpu as pltpu
from jax.experimental.pallas import tpu_sc as plsc

_SIZES = [44, 11, 11, 11, 11, 11, 6, 6, 5, 2, 2]
_NT = len(_SIZES)
_OFF = [sum(_SIZES[:i]) for i in range(_NT)]
_ROWS = sum(_SIZES)
_D = 128
_E = 320000
_NC, _NS, _L = 2, 16, 16
_NW = _NC * _NS
_EPW = _E // _NW
_C = 400
_GPC = _C // _L
_NCHUNK = _EPW // _C


def _body(x_hbm, w_hbm, out_hbm, wloc, xb, ob):
    cid = lax.axis_index("c")
    sid = lax.axis_index("s")
    wid = sid * _NC + cid
    pltpu.sync_copy(w_hbm, wloc)

    iota = lax.iota(jnp.int32, _L)
    stride_nt = iota * _NT
    odelta = iota * _D

    def chunk_body(k, _):
        e0 = wid * _EPW + k * _C
        pltpu.sync_copy(x_hbm.at[pl.ds(e0 * _NT, _C * _NT)], xb)

        def group_body(j, _):
            gbase = j * (_L * _NT)
            rowbase = []
            for g in range(_NT):
                iv = plsc.load_gather(xb, [stride_nt + (gbase + g)])
                rowbase.append((iv + _OFF[g]) * _D)
            obase = odelta + j * (_L * _D)

            def col_body(cb, _):
                c0 = cb * _L
                for u in range(_L):
                    c = c0 + u
                    acc = plsc.load_gather(wloc, [rowbase[0] + c])
                    for g in range(1, _NT):
                        acc = acc + plsc.load_gather(wloc, [rowbase[g] + c])
                    plsc.store_scatter(ob, [obase + c], acc)
                return 0

            lax.fori_loop(0, _D // _L, col_body, 0)
            return 0

        lax.fori_loop(0, _GPC, group_body, 0)
        pltpu.sync_copy(ob, out_hbm.at[pl.ds(e0 * _D, _C * _D)])
        return 0

    lax.fori_loop(0, _NCHUNK, chunk_body, 0)


@jax.jit
def _run(xflat, wcat):
    mesh = plsc.VectorSubcoreMesh(
        core_axis_name="c", subcore_axis_name="s",
        num_cores=_NC, num_subcores=_NS)
    kern = pl.kernel(
        _body,
        out_type=jax.ShapeDtypeStruct((_E * _D,), jnp.float32),
        mesh=mesh,
        scratch_types=[
            pltpu.VMEM((_ROWS * _D,), jnp.float32),
            pltpu.VMEM((_C * _NT,), jnp.int32),
            pltpu.VMEM((_C * _D,), jnp.float32),
        ],
        compiler_params=pltpu.CompilerParams(needs_layout_passes=False),
    )
    return kern(xflat, wcat)


def kernel(x, W0, W1, W2, W3, W4, W5, W6, W7, W8, W9, W10):
    wcat = jnp.concatenate(
        [W0, W1, W2, W3, W4, W5, W6, W7, W8, W9, W10], axis=0).reshape(-1)
    out = _run(x.reshape(-1), wcat)
    return out.reshape(_E, _D)


# speedup vs baseline: 8.8106x; 8.8106x over previous
import jax
import jax.numpy as jnp
from jax import lax
from jax.experimental import pallas as pl
from jax.experimental.pallas import tpu as pltpu
from jax.experimental.pallas import tpu_sc as plsc

_SIZES = [44, 11, 11, 11, 11, 11, 6, 6, 5, 2, 2]
_NT = len(_SIZES)
_OFF = [sum(_SIZES[:i]) for i in range(_NT)]
_D = 128
_E = 320000
_NC, _NS, _L = 2, 16, 16
_NW = _NC * _NS
_EPW = _E // _NW
_C = 400
_GPC = _C // _L
_NCHUNK = _EPW // _C

_GROUPS = [((0, 9, 10), 176), ((1, 2), 121), ((3, 4), 121),
           ((5, 6), 66), ((7, 8), 30)]
_GOFF = [0, 176, 297, 418, 484]
_ROWS_F = 514

_NCH = _D // _L


def _body(x_hbm, w_hbm, out_hbm, wtab, xb, ob):
    cid = lax.axis_index("c")
    sid = lax.axis_index("s")
    wid = sid * _NC + cid

    pltpu.sync_copy(w_hbm, ob.at[pl.ds(0, 120 * _D)])

    def _src(t):
        return _OFF[t] * _D

    t910 = 120 * _D

    def b910(a, _):
        av = [ob[pl.ds(_src(9) + a * _D + c * _L, _L)] for c in range(_NCH)]

        def bb(b, _):
            for c in range(_NCH):
                ob[pl.ds(t910 + (a * 2 + b) * _D + c * _L, _L)] = (
                    av[c] + ob[pl.ds(_src(10) + b * _D + c * _L, _L)])
            return 0

        lax.fori_loop(0, 2, bb, 0)
        return 0

    lax.fori_loop(0, 2, b910, 0)

    def build_pair(dst_off, a_off, sa, b_off, sb):
        def la(a, _):
            av = [ob[pl.ds(a_off + a * _D + c * _L, _L)] for c in range(_NCH)]
            dbase = (dst_off + a * sb) * _D

            def lb(b, _):
                for c in range(_NCH):
                    wtab[pl.ds(dbase + b * _D + c * _L, _L)] = (
                        av[c] + ob[pl.ds(b_off + b * _D + c * _L, _L)])
                return 0

            lax.fori_loop(0, sb, lb, 0)
            return 0

        lax.fori_loop(0, sa, la, 0)

    build_pair(_GOFF[0], _src(0), 44, t910, 4)
    build_pair(_GOFF[1], _src(1), 11, _src(2), 11)
    build_pair(_GOFF[2], _src(3), 11, _src(4), 11)
    build_pair(_GOFF[3], _src(5), 11, _src(6), 6)
    build_pair(_GOFF[4], _src(7), 6, _src(8), 5)

    iota = lax.iota(jnp.int32, _L)
    stride_nt = iota * _NT

    def chunk_body(k, _):
        e0 = wid * _EPW + k * _C
        pltpu.sync_copy(x_hbm.at[pl.ds(e0 * _NT, _C * _NT)], xb)

        def group_body(j, _):
            gb = j * (_L * _NT)
            iv = [plsc.load_gather(xb, [stride_nt + (gb + g)])
                  for g in range(_NT)]
            cidx = [
                (iv[0] * 2 + iv[9]) * 2 + iv[10],
                iv[1] * 11 + iv[2],
                iv[3] * 11 + iv[4],
                iv[5] * 6 + iv[6],
                iv[7] * 5 + iv[8],
            ]
            base = [(cidx[g] + _GOFF[g]) * _D for g in range(5)]
            for u in range(_L):
                b0 = base[0][u]
                b1 = base[1][u]
                b2 = base[2][u]
                b3 = base[3][u]
                b4 = base[4][u]
                o = (j * _L + u) * _D
                for c in range(_NCH):
                    cc = c * _L
                    acc = (wtab[pl.ds(b0 + cc, _L)]
                           + wtab[pl.ds(b1 + cc, _L)]
                           + wtab[pl.ds(b2 + cc, _L)]
                           + wtab[pl.ds(b3 + cc, _L)]
                           + wtab[pl.ds(b4 + cc, _L)])
                    ob[pl.ds(o + cc, _L)] = acc
            return 0

        lax.fori_loop(0, _GPC, group_body, 0)
        pltpu.sync_copy(ob, out_hbm.at[pl.ds(e0 * _D, _C * _D)])
        return 0

    lax.fori_loop(0, _NCHUNK, chunk_body, 0)


@jax.jit
def _run(xflat, wcat):
    mesh = plsc.VectorSubcoreMesh(
        core_axis_name="c", subcore_axis_name="s",
        num_cores=_NC, num_subcores=_NS)
    kern = pl.kernel(
        _body,
        out_type=jax.ShapeDtypeStruct((_E * _D,), jnp.float32),
        mesh=mesh,
        scratch_types=[
            pltpu.VMEM((_ROWS_F * _D,), jnp.float32),
            pltpu.VMEM((_C * _NT,), jnp.int32),
            pltpu.VMEM((_C * _D,), jnp.float32),
        ],
        compiler_params=pltpu.CompilerParams(needs_layout_passes=False),
    )
    return kern(xflat, wcat)


def kernel(x, W0, W1, W2, W3, W4, W5, W6, W7, W8, W9, W10):
    wcat = jnp.concatenate(
        [W0, W1, W2, W3, W4, W5, W6, W7, W8, W9, W10], axis=0).reshape(-1)
    out = _run(x.reshape(-1), wcat)
    return out.reshape(_E, _D)
